# baseline (speedup 1.0000x reference)
import functools

import jax
import jax.numpy as jnp
from jax import lax
from jax.experimental import pallas as pl
from jax.experimental.pallas import tpu as pltpu
from jax.experimental.pallas import tpu_sc as plsc

N = 10000
D = 128
K = 8
B = 16

NPAD = 10240
RB = 128
CB = 256
NRB = NPAD // RB
NCHUNK = NPAD // CB

R3 = 2000
NB3 = N // R3

NW = 32
EPW = 2560
EPAD = NW * EPW
GCHUNK = 128
NGC = EPW // GCHUNK

EPS = 1e-5
NEDGE = float(N * K)


def _sigmoid(v):
    return 1.0 / (1.0 + jnp.exp(-v))


def _knn_body(cstart_ref, ncb_ref, xr_ref, xfull_ref, brow_ref, bcol_ref,
              w1_ref, idx_ref, p_ref, q_ref):
    i = pl.program_id(0)
    csb = cstart_ref[i]
    ncb = ncb_ref[i]
    xr = xr_ref[...]
    brow = brow_ref[0]
    ones_row = jnp.ones((1, D), jnp.float32)
    rowsq = lax.dot_general(ones_row, xr * xr, (((1,), (1,)), ((), ())),
                            precision=lax.Precision.HIGHEST,
                            preferred_element_type=jnp.float32)

    def chunk_step(j, carry):
        bv, bi = carry
        cj = csb + j
        xc = xfull_ref[pl.ds(cj * CB, CB), :]
        colsq = jnp.sum(xc * xc, axis=1, keepdims=True)
        dots = lax.dot_general(xc, xr, (((1,), (1,)), ((), ())),
                               preferred_element_type=jnp.float32)
        sc = (rowsq - 2.0 * dots) + colsq
        bcol = bcol_ref[pl.ds(cj * CB, CB), :]
        sc = jnp.where(brow != bcol, jnp.inf, sc)
        cid0 = (cj * CB).astype(jnp.float32)
        iota_row = lax.broadcasted_iota(
            jnp.int32, (1, CB), 1).astype(jnp.float32)
        for _ in range(K):
            m = jnp.min(sc, axis=0, keepdims=True)
            eq = sc == m
            eqf = jnp.where(eq, 1.0, 0.0)
            am = lax.dot_general(iota_row, eqf, (((1,), (0,)), ((), ())),
                                 preferred_element_type=jnp.float32)
            am = jnp.minimum(am + cid0, float(NPAD - 1))
            sc = jnp.where(eq, jnp.inf, sc)
            mb = jnp.broadcast_to(m, (K, RB))
            ab = jnp.broadcast_to(am, (K, RB))
            keep = bv <= mb
            sh_v = jnp.concatenate([m, bv[:-1, :]], axis=0)
            sh_i = jnp.concatenate([am, bi[:-1, :]], axis=0)
            keep_prev = sh_v <= mb
            bv = jnp.where(keep, bv, jnp.where(keep_prev, mb, sh_v))
            bi = jnp.where(keep, bi, jnp.where(keep_prev, ab, sh_i))
        return bv, bi

    bv0 = jnp.full((K, RB), jnp.inf, jnp.float32)
    bi0 = jnp.zeros((K, RB), jnp.float32)
    _, bi = lax.fori_loop(0, ncb, chunk_step, (bv0, bi0))
    idx_ref[...] = bi.astype(jnp.int32)
    wa = w1_ref[:D, :] - w1_ref[D:, :]
    wb = w1_ref[D:, :]
    p_ref[...] = lax.dot_general(xr, wa, (((1,), (0,)), ((), ())),
                                 preferred_element_type=jnp.float32)
    q_ref[...] = lax.dot_general(xr, wb, (((1,), (0,)), ((), ())),
                                 preferred_element_type=jnp.float32)


def _knn_call(cstart, ncb, xpad, brow, bcol, w1, interpret=False):
    grid_spec = pltpu.PrefetchScalarGridSpec(
        num_scalar_prefetch=2,
        grid=(NRB,),
        in_specs=[
            pl.BlockSpec((RB, D), lambda i, *_: (i, 0)),
            pl.BlockSpec((NPAD, D), lambda i, *_: (0, 0)),
            pl.BlockSpec((1, 1, RB), lambda i, *_: (i, 0, 0)),
            pl.BlockSpec((NPAD, 1), lambda i, *_: (0, 0)),
            pl.BlockSpec((2 * D, D), lambda i, *_: (0, 0)),
        ],
        out_specs=[
            pl.BlockSpec((K, RB), lambda i, *_: (0, i)),
            pl.BlockSpec((RB, D), lambda i, *_: (i, 0)),
            pl.BlockSpec((RB, D), lambda i, *_: (i, 0)),
        ],
    )
    return pl.pallas_call(
        _knn_body,
        grid_spec=grid_spec,
        out_shape=[
            jax.ShapeDtypeStruct((K, NPAD), jnp.int32),
            jax.ShapeDtypeStruct((NPAD, D), jnp.float32),
            jax.ShapeDtypeStruct((NPAD, D), jnp.float32),
        ],
        interpret=interpret,
    )(cstart, ncb, xpad, xpad, brow, bcol, w1)


NBUF = 4


def _sc_gather(qtab, idx_flat):
    mesh = plsc.VectorSubcoreMesh(core_axis_name="c", subcore_axis_name="s")

    @functools.partial(
        pl.kernel,
        mesh=mesh,
        out_type=jax.ShapeDtypeStruct((EPAD, D), jnp.float32),
        scratch_types=[
            pltpu.VMEM((EPW,), jnp.int32),
            pltpu.VMEM((NBUF, GCHUNK, D), jnp.float32),
            pltpu.SemaphoreType.DMA((NBUF,)),
            pltpu.SemaphoreType.DMA((NBUF,)),
        ],
    )
    def gather_kernel(tab_hbm, idx_hbm, out_hbm, idx_v, rows_v, gsem, osem):
        wid = lax.axis_index("s") * 2 + lax.axis_index("c")
        base = wid * EPW
        pltpu.sync_copy(idx_hbm.at[pl.ds(base, EPW)], idx_v)

        hg = [None] * NGC
        ho = [None] * NGC
        for t in range(NGC):
            b = t % NBUF
            if t >= NBUF:
                ho[t - NBUF].wait()
            hg[t] = pltpu.async_copy(
                tab_hbm.at[idx_v.at[pl.ds(t * GCHUNK, GCHUNK)]],
                rows_v.at[b], gsem.at[b])
            if t >= 1:
                bp = (t - 1) % NBUF
                hg[t - 1].wait()
                ho[t - 1] = pltpu.async_copy(
                    rows_v.at[bp],
                    out_hbm.at[pl.ds(base + (t - 1) * GCHUNK, GCHUNK)],
                    osem.at[bp])
        hg[NGC - 1].wait()
        ho[NGC - 1] = pltpu.async_copy(
            rows_v.at[(NGC - 1) % NBUF],
            out_hbm.at[pl.ds(base + (NGC - 1) * GCHUNK, GCHUNK)],
            osem.at[(NGC - 1) % NBUF])
        for t in range(max(0, NGC - NBUF), NGC):
            ho[t].wait()

    return gather_kernel(qtab, idx_flat)


def _attn_body(cons_ref, qg_ref, p_ref, b1t_ref, g1t_ref, be1t_ref, s2w_ref,
               out_ref, acc_s, acc_q, mu_t, inv_t, acc_g, acc_g2, gstat):
    ph = pl.program_id(0)
    blk = pl.program_id(1)

    pk = p_ref[...]
    pt = jnp.concatenate([pk] * K, axis=1)
    y = qg_ref[...] + pt + b1t_ref[...]
    def colsum(v):
        return jnp.sum(v, axis=0, keepdims=True)

    @pl.when((ph == 0) & (blk == 0))
    def _init():
        acc_s[...] = jnp.zeros_like(acc_s)
        acc_q[...] = jnp.zeros_like(acc_q)
        acc_g[...] = jnp.zeros_like(acc_g)
        acc_g2[...] = jnp.zeros_like(acc_g2)

    @pl.when(ph == 0)
    def _phase0():
        acc_s[...] += colsum(y)
        acc_q[...] += colsum(y * y)
        out_ref[...] = jnp.zeros((R3, D), jnp.float32)

    @pl.when((ph == 1) & (blk == 0))
    def _finalize_bn1():
        s = acc_s[...]
        q = acc_q[...]
        s128 = jnp.zeros((1, D), jnp.float32)
        q128 = jnp.zeros((1, D), jnp.float32)
        for k in range(K):
            s128 += s[:, k * D:(k + 1) * D]
            q128 += q[:, k * D:(k + 1) * D]
        mu = s128 / NEDGE
        var = q128 / NEDGE - mu * mu
        inv = 1.0 / jnp.sqrt(var + EPS)
        mu_t[...] = jnp.concatenate([mu] * K, axis=1)
        inv_t[...] = jnp.concatenate([inv] * K, axis=1)

    def compute_mg():
        ym = (y - mu_t[...]) * inv_t[...] * g1t_ref[...] + be1t_ref[...]
        m = ym * _sigmoid(ym)
        mm = m * s2w_ref[...]
        g0 = jnp.concatenate(
            [jnp.sum(mm[:, k * D:(k + 1) * D], axis=1, keepdims=True)
             for k in range(K)], axis=1)
        g0 = g0 + cons_ref[0]
        return m, g0

    @pl.when(ph == 1)
    def _phase1():
        _, g0 = compute_mg()
        acc_g[...] += colsum(g0)
        acc_g2[...] += colsum(g0 * g0)
        out_ref[...] = jnp.zeros((R3, D), jnp.float32)

    @pl.when((ph == 2) & (blk == 0))
    def _finalize_bn2():
        mug = jnp.sum(acc_g[...], keepdims=True) / NEDGE
        varg = jnp.sum(acc_g2[...], keepdims=True) / NEDGE - mug * mug
        invg = 1.0 / jnp.sqrt(varg + EPS)
        gstat[...] = jnp.concatenate([mug, invg], axis=1)

    @pl.when(ph == 2)
    def _phase2():
        m, g0 = compute_mg()
        mug = gstat[0:1, 0:1]
        invg = gstat[0:1, 1:2]
        gp = (g0 - mug) * invg * cons_ref[1] + cons_ref[2]
        ga = gp * _sigmoid(gp)
        mx = jnp.max(ga, axis=1, keepdims=True)
        e = jnp.exp(ga - mx)
        w = e / jnp.sum(e, axis=1, keepdims=True)
        out = jnp.zeros((R3, D), jnp.float32)
        for k in range(K):
            out += w[:, k:k + 1] * m[:, k * D:(k + 1) * D]
        out_ref[...] = out


def _attn_call(cons, qg2, ppad, b1t, g1t, be1t, s2w, interpret=False):
    grid_spec = pltpu.PrefetchScalarGridSpec(
        num_scalar_prefetch=1,
        grid=(3, NB3),
        in_specs=[
            pl.BlockSpec((R3, K * D), lambda p, b, *_: (b, 0)),
            pl.BlockSpec((R3, D), lambda p, b, *_: (b, 0)),
            pl.BlockSpec((1, K * D), lambda p, b, *_: (0, 0)),
            pl.BlockSpec((1, K * D), lambda p, b, *_: (0, 0)),
            pl.BlockSpec((1, K * D), lambda p, b, *_: (0, 0)),
            pl.BlockSpec((1, K * D), lambda p, b, *_: (0, 0)),
        ],
        out_specs=pl.BlockSpec((R3, D), lambda p, b, *_: (b, 0)),
        scratch_shapes=[
            pltpu.VMEM((1, K * D), jnp.float32),
            pltpu.VMEM((1, K * D), jnp.float32),
            pltpu.VMEM((1, K * D), jnp.float32),
            pltpu.VMEM((1, K * D), jnp.float32),
            pltpu.VMEM((1, K), jnp.float32),
            pltpu.VMEM((1, K), jnp.float32),
            pltpu.VMEM((1, 2), jnp.float32),
        ],
    )
    return pl.pallas_call(
        _attn_body,
        grid_spec=grid_spec,
        out_shape=jax.ShapeDtypeStruct((N, D), jnp.float32),
        interpret=interpret,
    )(cons, qg2, ppad, b1t, g1t, be1t, s2w)


def _prep_knn_inputs(x, batch):
    xpad = jnp.concatenate(
        [x, jnp.zeros((NPAD - N, D), jnp.float32)], axis=0)
    bf = batch.astype(jnp.float32)
    brow = jnp.concatenate(
        [bf, jnp.full((NPAD - N,), -2.0, jnp.float32)]).reshape(NRB, 1, RB)
    bcol = jnp.concatenate(
        [bf, jnp.full((NPAD - N,), -1.0, jnp.float32)]).reshape(NPAD, 1)
    seg_lo = jnp.searchsorted(batch, jnp.arange(B, dtype=batch.dtype),
                              side="left")
    seg_hi = jnp.searchsorted(batch, jnp.arange(B, dtype=batch.dtype),
                              side="right")
    r0 = jnp.arange(NRB) * RB
    r1 = jnp.minimum(r0 + RB - 1, N - 1)
    blo = batch[jnp.minimum(r0, N - 1)]
    bhi = batch[r1]
    c0 = (seg_lo[blo] // CB).astype(jnp.int32)
    c1 = ((seg_hi[bhi] + CB - 1) // CB).astype(jnp.int32)
    ncb = jnp.where(r0 < N, c1 - c0, 0).astype(jnp.int32)
    return xpad, brow, bcol, c0, ncb


def kernel(x, batch, W1, b1, g1, be1, Wg, bg, gg, beg):
    xpad, brow, bcol, cstart, ncb = _prep_knn_inputs(x, batch)
    idx_t, ppad, qpad = _knn_call(cstart, ncb, xpad, brow, bcol, W1)
    idx = idx_t.T

    idx_flat = idx[:N].reshape(N * K)
    idx_flat = jnp.concatenate(
        [idx_flat, jnp.zeros((EPAD - N * K,), jnp.int32)])
    qg = _sc_gather(qpad, idx_flat)
    qg2 = qg.reshape(NPAD, K * D)

    cons = jnp.stack([bg[0], gg[0], beg[0]]).astype(jnp.float32)
    b1t = jnp.tile(b1, K).reshape(1, K * D)
    g1t = jnp.tile(g1, K).reshape(1, K * D)
    be1t = jnp.tile(be1, K).reshape(1, K * D)
    wgt = jnp.tile(Wg[:, 0], K).reshape(1, K * D)
    return _attn_call(cons, qg2, ppad, b1t, g1t, be1t, wgt)


# speedup vs baseline: 1.0307x; 1.0307x over previous
import functools

import jax
import jax.numpy as jnp
from jax import lax
from jax.experimental import pallas as pl
from jax.experimental.pallas import tpu as pltpu
from jax.experimental.pallas import tpu_sc as plsc

N = 10000
D = 128
K = 8
B = 16

NPAD = 10240
RB = 128
CB = 256
NRB = NPAD // RB
NCHUNK = NPAD // CB

R3 = 2000
NB3 = N // R3

NW = 32
EPW = 2560
EPAD = NW * EPW
GCHUNK = 128
NGC = EPW // GCHUNK

EPS = 1e-5
NEDGE = float(N * K)


def _sigmoid(v):
    return 1.0 / (1.0 + jnp.exp(-v))


def _knn_body(cstart_ref, ncb_ref, xr_ref, xfull_ref, brow_ref, bcol_ref,
              w1_ref, idx_ref, p_ref, q_ref):
    i = pl.program_id(0)
    csb = cstart_ref[i]
    ncb = ncb_ref[i]
    xr = xr_ref[...]
    brow = brow_ref[0]
    ones_row = jnp.ones((1, D), jnp.float32)
    rowsq = lax.dot_general(ones_row, xr * xr, (((1,), (1,)), ((), ())),
                            precision=lax.Precision.HIGHEST,
                            preferred_element_type=jnp.float32)

    def chunk_step(j, carry):
        bv, bi = carry
        cj = csb + j
        xc = xfull_ref[pl.ds(cj * CB, CB), :]
        colsq = jnp.sum(xc * xc, axis=1, keepdims=True)
        dots = lax.dot_general(xc, xr, (((1,), (1,)), ((), ())),
                               preferred_element_type=jnp.float32)
        sc = (rowsq - 2.0 * dots) + colsq
        bcol = bcol_ref[pl.ds(cj * CB, CB), :]
        sc = jnp.where(brow != bcol, jnp.inf, sc)
        cid0 = (cj * CB).astype(jnp.float32)
        srcid = cid0 + lax.broadcasted_iota(
            jnp.int32, (CB, 1), 0).astype(jnp.float32)
        srcid_b = jnp.broadcast_to(srcid, (CB, RB))
        for _ in range(K):
            m = jnp.min(sc, axis=0, keepdims=True)
            eq = sc == m
            am = jnp.min(jnp.where(eq, srcid_b, 1e9),
                         axis=0, keepdims=True)
            sc = jnp.where(eq, jnp.inf, sc)
            mb = jnp.broadcast_to(m, (K, RB))
            ab = jnp.broadcast_to(am, (K, RB))
            keep = bv <= mb
            sh_v = jnp.concatenate([m, bv[:-1, :]], axis=0)
            sh_i = jnp.concatenate([am, bi[:-1, :]], axis=0)
            keep_prev = sh_v <= mb
            bv = jnp.where(keep, bv, jnp.where(keep_prev, mb, sh_v))
            bi = jnp.where(keep, bi, jnp.where(keep_prev, ab, sh_i))
        return bv, bi

    bv0 = jnp.full((K, RB), jnp.inf, jnp.float32)
    bi0 = jnp.zeros((K, RB), jnp.float32)
    _, bi = lax.fori_loop(0, ncb, chunk_step, (bv0, bi0))
    idx_ref[...] = bi.astype(jnp.int32)
    wa = w1_ref[:D, :] - w1_ref[D:, :]
    wb = w1_ref[D:, :]
    p_ref[...] = lax.dot_general(xr, wa, (((1,), (0,)), ((), ())),
                                 preferred_element_type=jnp.float32)
    q_ref[...] = lax.dot_general(xr, wb, (((1,), (0,)), ((), ())),
                                 preferred_element_type=jnp.float32)


def _knn_call(cstart, ncb, xpad, brow, bcol, w1, interpret=False):
    grid_spec = pltpu.PrefetchScalarGridSpec(
        num_scalar_prefetch=2,
        grid=(NRB,),
        in_specs=[
            pl.BlockSpec((RB, D), lambda i, *_: (i, 0)),
            pl.BlockSpec((NPAD, D), lambda i, *_: (0, 0)),
            pl.BlockSpec((1, 1, RB), lambda i, *_: (i, 0, 0)),
            pl.BlockSpec((NPAD, 1), lambda i, *_: (0, 0)),
            pl.BlockSpec((2 * D, D), lambda i, *_: (0, 0)),
        ],
        out_specs=[
            pl.BlockSpec((K, RB), lambda i, *_: (0, i)),
            pl.BlockSpec((RB, D), lambda i, *_: (i, 0)),
            pl.BlockSpec((RB, D), lambda i, *_: (i, 0)),
        ],
    )
    return pl.pallas_call(
        _knn_body,
        grid_spec=grid_spec,
        out_shape=[
            jax.ShapeDtypeStruct((K, NPAD), jnp.int32),
            jax.ShapeDtypeStruct((NPAD, D), jnp.float32),
            jax.ShapeDtypeStruct((NPAD, D), jnp.float32),
        ],
        interpret=interpret,
    )(cstart, ncb, xpad, xpad, brow, bcol, w1)


NBUF = 4


def _sc_gather(qtab, idx_flat):
    mesh = plsc.VectorSubcoreMesh(core_axis_name="c", subcore_axis_name="s")

    @functools.partial(
        pl.kernel,
        mesh=mesh,
        out_type=jax.ShapeDtypeStruct((EPAD, D), jnp.float32),
        scratch_types=[
            pltpu.VMEM((EPW,), jnp.int32),
            pltpu.VMEM((NBUF, GCHUNK, D), jnp.float32),
            pltpu.SemaphoreType.DMA((NBUF,)),
            pltpu.SemaphoreType.DMA((NBUF,)),
        ],
    )
    def gather_kernel(tab_hbm, idx_hbm, out_hbm, idx_v, rows_v, gsem, osem):
        wid = lax.axis_index("s") * 2 + lax.axis_index("c")
        base = wid * EPW
        pltpu.sync_copy(idx_hbm.at[pl.ds(base, EPW)], idx_v)

        hg = [None] * NGC
        ho = [None] * NGC
        for t in range(NGC):
            b = t % NBUF
            if t >= NBUF:
                ho[t - NBUF].wait()
            hg[t] = pltpu.async_copy(
                tab_hbm.at[idx_v.at[pl.ds(t * GCHUNK, GCHUNK)]],
                rows_v.at[b], gsem.at[b])
            if t >= 1:
                bp = (t - 1) % NBUF
                hg[t - 1].wait()
                ho[t - 1] = pltpu.async_copy(
                    rows_v.at[bp],
                    out_hbm.at[pl.ds(base + (t - 1) * GCHUNK, GCHUNK)],
                    osem.at[bp])
        hg[NGC - 1].wait()
        ho[NGC - 1] = pltpu.async_copy(
            rows_v.at[(NGC - 1) % NBUF],
            out_hbm.at[pl.ds(base + (NGC - 1) * GCHUNK, GCHUNK)],
            osem.at[(NGC - 1) % NBUF])
        for t in range(max(0, NGC - NBUF), NGC):
            ho[t].wait()

    return gather_kernel(qtab, idx_flat)


def _attn_body(cons_ref, qg_ref, p_ref, b1t_ref, g1t_ref, be1t_ref, s2w_ref,
               out_ref, acc_s, acc_q, mu_t, inv_t, acc_g, acc_g2, gstat):
    ph = pl.program_id(0)
    blk = pl.program_id(1)

    pk = p_ref[...]
    pt = jnp.concatenate([pk] * K, axis=1)
    y = qg_ref[...] + pt + b1t_ref[...]
    def colsum(v):
        return jnp.sum(v, axis=0, keepdims=True)

    @pl.when((ph == 0) & (blk == 0))
    def _init():
        acc_s[...] = jnp.zeros_like(acc_s)
        acc_q[...] = jnp.zeros_like(acc_q)
        acc_g[...] = jnp.zeros_like(acc_g)
        acc_g2[...] = jnp.zeros_like(acc_g2)

    @pl.when(ph == 0)
    def _phase0():
        acc_s[...] += colsum(y)
        acc_q[...] += colsum(y * y)
        out_ref[...] = jnp.zeros((R3, D), jnp.float32)

    @pl.when((ph == 1) & (blk == 0))
    def _finalize_bn1():
        s = acc_s[...]
        q = acc_q[...]
        s128 = jnp.zeros((1, D), jnp.float32)
        q128 = jnp.zeros((1, D), jnp.float32)
        for k in range(K):
            s128 += s[:, k * D:(k + 1) * D]
            q128 += q[:, k * D:(k + 1) * D]
        mu = s128 / NEDGE
        var = q128 / NEDGE - mu * mu
        inv = 1.0 / jnp.sqrt(var + EPS)
        mu_t[...] = jnp.concatenate([mu] * K, axis=1)
        inv_t[...] = jnp.concatenate([inv] * K, axis=1)

    def compute_mg():
        ym = (y - mu_t[...]) * inv_t[...] * g1t_ref[...] + be1t_ref[...]
        m = ym * _sigmoid(ym)
        mm = m * s2w_ref[...]
        g0 = jnp.concatenate(
            [jnp.sum(mm[:, k * D:(k + 1) * D], axis=1, keepdims=True)
             for k in range(K)], axis=1)
        g0 = g0 + cons_ref[0]
        return m, g0

    @pl.when(ph == 1)
    def _phase1():
        _, g0 = compute_mg()
        acc_g[...] += colsum(g0)
        acc_g2[...] += colsum(g0 * g0)
        out_ref[...] = jnp.zeros((R3, D), jnp.float32)

    @pl.when((ph == 2) & (blk == 0))
    def _finalize_bn2():
        mug = jnp.sum(acc_g[...], keepdims=True) / NEDGE
        varg = jnp.sum(acc_g2[...], keepdims=True) / NEDGE - mug * mug
        invg = 1.0 / jnp.sqrt(varg + EPS)
        gstat[...] = jnp.concatenate([mug, invg], axis=1)

    @pl.when(ph == 2)
    def _phase2():
        m, g0 = compute_mg()
        mug = gstat[0:1, 0:1]
        invg = gstat[0:1, 1:2]
        gp = (g0 - mug) * invg * cons_ref[1] + cons_ref[2]
        ga = gp * _sigmoid(gp)
        mx = jnp.max(ga, axis=1, keepdims=True)
        e = jnp.exp(ga - mx)
        w = e / jnp.sum(e, axis=1, keepdims=True)
        out = jnp.zeros((R3, D), jnp.float32)
        for k in range(K):
            out += w[:, k:k + 1] * m[:, k * D:(k + 1) * D]
        out_ref[...] = out


def _attn_call(cons, qg2, ppad, b1t, g1t, be1t, s2w, interpret=False):
    grid_spec = pltpu.PrefetchScalarGridSpec(
        num_scalar_prefetch=1,
        grid=(3, NB3),
        in_specs=[
            pl.BlockSpec((R3, K * D), lambda p, b, *_: (b, 0)),
            pl.BlockSpec((R3, D), lambda p, b, *_: (b, 0)),
            pl.BlockSpec((1, K * D), lambda p, b, *_: (0, 0)),
            pl.BlockSpec((1, K * D), lambda p, b, *_: (0, 0)),
            pl.BlockSpec((1, K * D), lambda p, b, *_: (0, 0)),
            pl.BlockSpec((1, K * D), lambda p, b, *_: (0, 0)),
        ],
        out_specs=pl.BlockSpec((R3, D), lambda p, b, *_: (b, 0)),
        scratch_shapes=[
            pltpu.VMEM((1, K * D), jnp.float32),
            pltpu.VMEM((1, K * D), jnp.float32),
            pltpu.VMEM((1, K * D), jnp.float32),
            pltpu.VMEM((1, K * D), jnp.float32),
            pltpu.VMEM((1, K), jnp.float32),
            pltpu.VMEM((1, K), jnp.float32),
            pltpu.VMEM((1, 2), jnp.float32),
        ],
    )
    return pl.pallas_call(
        _attn_body,
        grid_spec=grid_spec,
        out_shape=jax.ShapeDtypeStruct((N, D), jnp.float32),
        interpret=interpret,
    )(cons, qg2, ppad, b1t, g1t, be1t, s2w)


def _prep_knn_inputs(x, batch):
    xpad = jnp.concatenate(
        [x, jnp.zeros((NPAD - N, D), jnp.float32)], axis=0)
    bf = batch.astype(jnp.float32)
    brow = jnp.concatenate(
        [bf, jnp.full((NPAD - N,), -2.0, jnp.float32)]).reshape(NRB, 1, RB)
    bcol = jnp.concatenate(
        [bf, jnp.full((NPAD - N,), -1.0, jnp.float32)]).reshape(NPAD, 1)
    seg_lo = jnp.searchsorted(batch, jnp.arange(B, dtype=batch.dtype),
                              side="left")
    seg_hi = jnp.searchsorted(batch, jnp.arange(B, dtype=batch.dtype),
                              side="right")
    r0 = jnp.arange(NRB) * RB
    r1 = jnp.minimum(r0 + RB - 1, N - 1)
    blo = batch[jnp.minimum(r0, N - 1)]
    bhi = batch[r1]
    c0 = (seg_lo[blo] // CB).astype(jnp.int32)
    c1 = ((seg_hi[bhi] + CB - 1) // CB).astype(jnp.int32)
    ncb = jnp.where(r0 < N, c1 - c0, 0).astype(jnp.int32)
    return xpad, brow, bcol, c0, ncb


def kernel(x, batch, W1, b1, g1, be1, Wg, bg, gg, beg):
    xpad, brow, bcol, cstart, ncb = _prep_knn_inputs(x, batch)
    idx_t, ppad, qpad = _knn_call(cstart, ncb, xpad, brow, bcol, W1)
    idx = idx_t.T

    idx_flat = idx[:N].reshape(N * K)
    idx_flat = jnp.concatenate(
        [idx_flat, jnp.zeros((EPAD - N * K,), jnp.int32)])
    qg = _sc_gather(qpad, idx_flat)
    qg2 = qg.reshape(NPAD, K * D)

    cons = jnp.stack([bg[0], gg[0], beg[0]]).astype(jnp.float32)
    b1t = jnp.tile(b1, K).reshape(1, K * D)
    g1t = jnp.tile(g1, K).reshape(1, K * D)
    be1t = jnp.tile(be1, K).reshape(1, K * D)
    wgt = jnp.tile(Wg[:, 0], K).reshape(1, K * D)
    return _attn_call(cons, qg2, ppad, b1t, g1t, be1t, wgt)


# speedup vs baseline: 1.0492x; 1.0180x over previous
import functools

import jax
import jax.numpy as jnp
from jax import lax
from jax.experimental import pallas as pl
from jax.experimental.pallas import tpu as pltpu
from jax.experimental.pallas import tpu_sc as plsc

N = 10000
D = 128
K = 8
B = 16

NPAD = 10240
RB = 128
CB = 256
NRB = NPAD // RB
NCHUNK = NPAD // CB

R3 = 2000
NB3 = N // R3

NW = 32
EPW = 2560
EPAD = NW * EPW
GCHUNK = 128
NGC = EPW // GCHUNK

EPS = 1e-5
NEDGE = float(N * K)


def _sigmoid(v):
    return 1.0 / (1.0 + jnp.exp(-v))


def _knn_body(cstart_ref, ncb_ref, xr_ref, xfull_ref, brow_ref, bcol_ref,
              w1_ref, idx_ref, p_ref, q_ref):
    i = pl.program_id(0)
    csb = cstart_ref[i]
    ncb = ncb_ref[i]
    xr = xr_ref[...]
    brow = brow_ref[0]
    ones_row = jnp.ones((1, D), jnp.float32)
    rowsq = lax.dot_general(ones_row, xr * xr, (((1,), (1,)), ((), ())),
                            precision=lax.Precision.HIGHEST,
                            preferred_element_type=jnp.float32)

    def chunk_step(j, carry):
        bv, bi = carry
        cj = csb + j
        xc = xfull_ref[pl.ds(cj * CB, CB), :]
        colsq = jnp.sum(xc * xc, axis=1, keepdims=True)
        dots = lax.dot_general(xc, xr, (((1,), (1,)), ((), ())),
                               preferred_element_type=jnp.float32)
        sc = (rowsq - 2.0 * dots) + colsq
        bcol = bcol_ref[pl.ds(cj * CB, CB), :]
        sc = jnp.where(brow != bcol, jnp.inf, sc)
        cid0 = (cj * CB).astype(jnp.float32)
        srcid = cid0 + lax.broadcasted_iota(
            jnp.int32, (CB, 1), 0).astype(jnp.float32)
        srcid_b = jnp.broadcast_to(srcid, (CB, RB))
        for _ in range(K):
            m = jnp.min(sc, axis=0, keepdims=True)
            eq = sc == m
            am = jnp.min(jnp.where(eq, srcid_b, 1e9),
                         axis=0, keepdims=True)
            sc = jnp.where(eq, jnp.inf, sc)
            mb = jnp.broadcast_to(m, (K, RB))
            ab = jnp.broadcast_to(am, (K, RB))
            keep = bv <= mb
            sh_v = jnp.concatenate([m, bv[:-1, :]], axis=0)
            sh_i = jnp.concatenate([am, bi[:-1, :]], axis=0)
            keep_prev = sh_v <= mb
            bv = jnp.where(keep, bv, jnp.where(keep_prev, mb, sh_v))
            bi = jnp.where(keep, bi, jnp.where(keep_prev, ab, sh_i))
        return bv, bi

    bv0 = jnp.full((K, RB), jnp.inf, jnp.float32)
    bi0 = jnp.zeros((K, RB), jnp.float32)
    _, bi = lax.fori_loop(0, ncb, chunk_step, (bv0, bi0))
    idx_ref[...] = bi.astype(jnp.int32)
    wa = w1_ref[:D, :] - w1_ref[D:, :]
    wb = w1_ref[D:, :]
    p_ref[...] = lax.dot_general(xr, wa, (((1,), (0,)), ((), ())),
                                 preferred_element_type=jnp.float32)
    q_ref[...] = lax.dot_general(xr, wb, (((1,), (0,)), ((), ())),
                                 preferred_element_type=jnp.float32)


def _knn_call(cstart, ncb, xpad, brow, bcol, w1, interpret=False):
    grid_spec = pltpu.PrefetchScalarGridSpec(
        num_scalar_prefetch=2,
        grid=(NRB,),
        in_specs=[
            pl.BlockSpec((RB, D), lambda i, *_: (i, 0)),
            pl.BlockSpec((NPAD, D), lambda i, *_: (0, 0)),
            pl.BlockSpec((1, 1, RB), lambda i, *_: (i, 0, 0)),
            pl.BlockSpec((NPAD, 1), lambda i, *_: (0, 0)),
            pl.BlockSpec((2 * D, D), lambda i, *_: (0, 0)),
        ],
        out_specs=[
            pl.BlockSpec((K, RB), lambda i, *_: (0, i)),
            pl.BlockSpec((RB, D), lambda i, *_: (i, 0)),
            pl.BlockSpec((RB, D), lambda i, *_: (i, 0)),
        ],
    )
    return pl.pallas_call(
        _knn_body,
        grid_spec=grid_spec,
        out_shape=[
            jax.ShapeDtypeStruct((K, NPAD), jnp.int32),
            jax.ShapeDtypeStruct((NPAD, D), jnp.float32),
            jax.ShapeDtypeStruct((NPAD, D), jnp.float32),
        ],
        interpret=interpret,
    )(cstart, ncb, xpad, xpad, brow, bcol, w1)


NBUF = 4


def _sc_gather(qtab, idx_flat):
    mesh = plsc.VectorSubcoreMesh(core_axis_name="c", subcore_axis_name="s")

    @functools.partial(
        pl.kernel,
        mesh=mesh,
        out_type=jax.ShapeDtypeStruct((EPAD, D), jnp.float32),
        scratch_types=[
            pltpu.VMEM((EPW,), jnp.int32),
            pltpu.VMEM((NBUF, GCHUNK, D), jnp.float32),
            pltpu.SemaphoreType.DMA((NBUF,)),
            pltpu.SemaphoreType.DMA((NBUF,)),
        ],
    )
    def gather_kernel(tab_hbm, idx_hbm, out_hbm, idx_v, rows_v, gsem, osem):
        wid = lax.axis_index("s") * 2 + lax.axis_index("c")
        base = wid * EPW
        pltpu.sync_copy(idx_hbm.at[pl.ds(base, EPW)], idx_v)

        LAG = NBUF - 1
        hg = [None] * NGC
        ho = [None] * NGC

        def start_out(t):
            hg[t].wait()
            ho[t] = pltpu.async_copy(
                rows_v.at[t % NBUF],
                out_hbm.at[pl.ds(base + t * GCHUNK, GCHUNK)],
                osem.at[t % NBUF])

        for t in range(NGC):
            b = t % NBUF
            if t >= NBUF:
                ho[t - NBUF].wait()
            hg[t] = pltpu.async_copy(
                tab_hbm.at[idx_v.at[pl.ds(t * GCHUNK, GCHUNK)]],
                rows_v.at[b], gsem.at[b])
            if t >= LAG:
                start_out(t - LAG)
        for t in range(max(0, NGC - LAG), NGC):
            start_out(t)
        for t in range(max(0, NGC - NBUF), NGC):
            ho[t].wait()

    return gather_kernel(qtab, idx_flat)


def _attn_body(cons_ref, qg_ref, p_ref, g1t_ref, be1t_ref, s2w_ref,
               out_ref, acc_s, acc_q, mu_t, inv_t, acc_g, acc_g2, gstat):
    ph = pl.program_id(0)
    blk = pl.program_id(1)

    pk = p_ref[...]
    pt = jnp.concatenate([pk] * K, axis=1)
    y = qg_ref[...] + pt
    def colsum(v):
        return jnp.sum(v, axis=0, keepdims=True)

    @pl.when((ph == 0) & (blk == 0))
    def _init():
        acc_s[...] = jnp.zeros_like(acc_s)
        acc_q[...] = jnp.zeros_like(acc_q)
        acc_g[...] = jnp.zeros_like(acc_g)
        acc_g2[...] = jnp.zeros_like(acc_g2)

    @pl.when(ph == 0)
    def _phase0():
        acc_s[...] += colsum(y)
        acc_q[...] += colsum(y * y)
        out_ref[...] = jnp.zeros((R3, D), jnp.float32)

    @pl.when((ph == 1) & (blk == 0))
    def _finalize_bn1():
        s = acc_s[...]
        q = acc_q[...]
        s128 = jnp.zeros((1, D), jnp.float32)
        q128 = jnp.zeros((1, D), jnp.float32)
        for k in range(K):
            s128 += s[:, k * D:(k + 1) * D]
            q128 += q[:, k * D:(k + 1) * D]
        mu = s128 / NEDGE
        var = q128 / NEDGE - mu * mu
        inv = 1.0 / jnp.sqrt(var + EPS)
        mu8 = jnp.concatenate([mu] * K, axis=1)
        inv8 = jnp.concatenate([inv] * K, axis=1)
        a = inv8 * g1t_ref[...]
        mu_t[...] = a
        inv_t[...] = be1t_ref[...] - mu8 * a

    def compute_mg():
        ym = y * mu_t[...] + inv_t[...]
        m = ym * _sigmoid(ym)
        mm = m * s2w_ref[...]
        g0 = jnp.concatenate(
            [jnp.sum(mm[:, k * D:(k + 1) * D], axis=1, keepdims=True)
             for k in range(K)], axis=1)
        g0 = g0 + cons_ref[0]
        return m, g0

    @pl.when(ph == 1)
    def _phase1():
        _, g0 = compute_mg()
        acc_g[...] += colsum(g0)
        acc_g2[...] += colsum(g0 * g0)
        out_ref[...] = jnp.zeros((R3, D), jnp.float32)

    @pl.when((ph == 2) & (blk == 0))
    def _finalize_bn2():
        mug = jnp.sum(acc_g[...], keepdims=True) / NEDGE
        varg = jnp.sum(acc_g2[...], keepdims=True) / NEDGE - mug * mug
        invg = 1.0 / jnp.sqrt(varg + EPS)
        gstat[...] = jnp.concatenate([mug, invg], axis=1)

    @pl.when(ph == 2)
    def _phase2():
        m, g0 = compute_mg()
        mug = gstat[0:1, 0:1]
        invg = gstat[0:1, 1:2]
        gp = (g0 - mug) * invg * cons_ref[1] + cons_ref[2]
        ga = gp * _sigmoid(gp)
        mx = jnp.max(ga, axis=1, keepdims=True)
        e = jnp.exp(ga - mx)
        w = e / jnp.sum(e, axis=1, keepdims=True)
        out = jnp.zeros((R3, D), jnp.float32)
        for k in range(K):
            out += w[:, k:k + 1] * m[:, k * D:(k + 1) * D]
        out_ref[...] = out


def _attn_call(cons, qg2, ppad, g1t, be1t, s2w, interpret=False):
    grid_spec = pltpu.PrefetchScalarGridSpec(
        num_scalar_prefetch=1,
        grid=(3, NB3),
        in_specs=[
            pl.BlockSpec((R3, K * D), lambda p, b, *_: (b, 0)),
            pl.BlockSpec((R3, D), lambda p, b, *_: (b, 0)),
            pl.BlockSpec((1, K * D), lambda p, b, *_: (0, 0)),
            pl.BlockSpec((1, K * D), lambda p, b, *_: (0, 0)),
            pl.BlockSpec((1, K * D), lambda p, b, *_: (0, 0)),
        ],
        out_specs=pl.BlockSpec((R3, D), lambda p, b, *_: (b, 0)),
        scratch_shapes=[
            pltpu.VMEM((1, K * D), jnp.float32),
            pltpu.VMEM((1, K * D), jnp.float32),
            pltpu.VMEM((1, K * D), jnp.float32),
            pltpu.VMEM((1, K * D), jnp.float32),
            pltpu.VMEM((1, K), jnp.float32),
            pltpu.VMEM((1, K), jnp.float32),
            pltpu.VMEM((1, 2), jnp.float32),
        ],
    )
    return pl.pallas_call(
        _attn_body,
        grid_spec=grid_spec,
        out_shape=jax.ShapeDtypeStruct((N, D), jnp.float32),
        interpret=interpret,
    )(cons, qg2, ppad, g1t, be1t, s2w)


def _prep_knn_inputs(x, batch):
    xpad = jnp.concatenate(
        [x, jnp.zeros((NPAD - N, D), jnp.float32)], axis=0)
    bf = batch.astype(jnp.float32)
    brow = jnp.concatenate(
        [bf, jnp.full((NPAD - N,), -2.0, jnp.float32)]).reshape(NRB, 1, RB)
    bcol = jnp.concatenate(
        [bf, jnp.full((NPAD - N,), -1.0, jnp.float32)]).reshape(NPAD, 1)
    seg_lo = jnp.searchsorted(batch, jnp.arange(B, dtype=batch.dtype),
                              side="left")
    seg_hi = jnp.searchsorted(batch, jnp.arange(B, dtype=batch.dtype),
                              side="right")
    r0 = jnp.arange(NRB) * RB
    r1 = jnp.minimum(r0 + RB - 1, N - 1)
    blo = batch[jnp.minimum(r0, N - 1)]
    bhi = batch[r1]
    c0 = (seg_lo[blo] // CB).astype(jnp.int32)
    c1 = ((seg_hi[bhi] + CB - 1) // CB).astype(jnp.int32)
    ncb = jnp.where(r0 < N, c1 - c0, 0).astype(jnp.int32)
    return xpad, brow, bcol, c0, ncb


def kernel(x, batch, W1, b1, g1, be1, Wg, bg, gg, beg):
    xpad, brow, bcol, cstart, ncb = _prep_knn_inputs(x, batch)
    idx_t, ppad, qpad = _knn_call(cstart, ncb, xpad, brow, bcol, W1)
    idx = idx_t.T

    idx_flat = idx[:N].reshape(N * K)
    idx_flat = jnp.concatenate(
        [idx_flat, jnp.zeros((EPAD - N * K,), jnp.int32)])
    qg = _sc_gather(qpad, idx_flat)
    qg2 = qg.reshape(NPAD, K * D)

    cons = jnp.stack([bg[0], gg[0], beg[0]]).astype(jnp.float32)
    del b1
    g1t = jnp.tile(g1, K).reshape(1, K * D)
    be1t = jnp.tile(be1, K).reshape(1, K * D)
    wgt = jnp.tile(Wg[:, 0], K).reshape(1, K * D)
    return _attn_call(cons, qg2, ppad, g1t, be1t, wgt)
